# Initial kernel scaffold; baseline (speedup 1.0000x reference)
import jax
import jax.numpy as jnp
from jax.experimental import pallas as pl


def kernel(x, dist_mat, W, b):
    raise NotImplementedError("write your pallas kernel here")


# speedup vs baseline: 1.6547x; 1.6547x over previous
import functools

import jax
import jax.numpy as jnp
from jax.experimental import pallas as pl

MAP_UNITS = 2048
IN_CH = 256
BATCH = 8
DIST_THRESHOLD = 200.0
ROW_BLK = 256
N_BLOCKS = MAP_UNITS // ROW_BLK


def _pass_a_kernel(d_ref, x_ref, wc_ref, deg_ref, u_ref, z_ref):
    d = d_ref[...]
    within = (d > 0.0) & (d < DIST_THRESHOLD)
    edge = jnp.where(within, d, 0.0)
    deg_ref[...] = jnp.sum(edge, axis=1)[None, :]

    x = x_ref[...]
    wc = wc_ref[...]
    zu = jax.lax.dot_general(
        x, wc, (((2,), (0,)), ((), ())),
        preferred_element_type=jnp.float32)
    u_ref[...] = zu[:, :, 0]
    z_ref[...] = zu[:, :, 1]


def _pass_b_kernel(d_ref, x_ref, deg_ref, u_ref, z_ref, b_ref, y_ref):
    i = pl.program_id(0)
    d = d_ref[...]
    within = (d > 0.0) & (d < DIST_THRESHOLD)
    edge = jnp.where(within, d, 0.0)

    deg = deg_ref[...]
    dinv = jnp.where(deg > 0.0, jax.lax.rsqrt(deg), 0.0)
    w = z_ref[...] * dinv
    t = jax.lax.dot_general(
        w, edge, (((1,), (1,)), ((), ())),
        preferred_element_type=jnp.float32)
    deg_n = deg_ref[0, pl.ds(i * ROW_BLK, ROW_BLK)]
    dinv_n = jnp.where(deg_n > 0.0, jax.lax.rsqrt(deg_n), 0.0)
    u = u_ref[:, pl.ds(i * ROW_BLK, ROW_BLK)]
    out = u - dinv_n[None, :] * t + b_ref[0, 0]
    gcn = jax.nn.sigmoid(out)

    y_ref[:, :, 0:IN_CH] = x_ref[...]
    y_ref[:, :, IN_CH:IN_CH + 1] = gcn[:, :, None]


@jax.jit
def kernel(x, dist_mat, W, b):
    wc = jnp.concatenate([W[0], W[1]], axis=1)
    b2 = jnp.reshape(b, (1, 1)).astype(jnp.float32)

    deg, u, z = pl.pallas_call(
        _pass_a_kernel,
        grid=(N_BLOCKS,),
        in_specs=[
            pl.BlockSpec((ROW_BLK, MAP_UNITS), lambda i: (i, 0)),
            pl.BlockSpec((BATCH, ROW_BLK, IN_CH), lambda i: (0, i, 0)),
            pl.BlockSpec((IN_CH, 2), lambda i: (0, 0)),
        ],
        out_specs=[
            pl.BlockSpec((1, ROW_BLK), lambda i: (0, i)),
            pl.BlockSpec((BATCH, ROW_BLK), lambda i: (0, i)),
            pl.BlockSpec((BATCH, ROW_BLK), lambda i: (0, i)),
        ],
        out_shape=[
            jax.ShapeDtypeStruct((1, MAP_UNITS), jnp.float32),
            jax.ShapeDtypeStruct((BATCH, MAP_UNITS), jnp.float32),
            jax.ShapeDtypeStruct((BATCH, MAP_UNITS), jnp.float32),
        ],
    )(dist_mat, x, wc)

    y = pl.pallas_call(
        _pass_b_kernel,
        grid=(N_BLOCKS,),
        in_specs=[
            pl.BlockSpec((ROW_BLK, MAP_UNITS), lambda i: (i, 0)),
            pl.BlockSpec((BATCH, ROW_BLK, IN_CH), lambda i: (0, i, 0)),
            pl.BlockSpec((1, MAP_UNITS), lambda i: (0, 0)),
            pl.BlockSpec((BATCH, MAP_UNITS), lambda i: (0, 0)),
            pl.BlockSpec((BATCH, MAP_UNITS), lambda i: (0, 0)),
            pl.BlockSpec((1, 1), lambda i: (0, 0)),
        ],
        out_specs=pl.BlockSpec((BATCH, ROW_BLK, IN_CH + 1), lambda i: (0, i, 0)),
        out_shape=jax.ShapeDtypeStruct(
            (BATCH, MAP_UNITS, IN_CH + 1), jnp.float32),
    )(dist_mat, x, deg, u, z, b2)

    return y


# speedup vs baseline: 1.9845x; 1.1993x over previous
import jax
import jax.numpy as jnp
from jax.experimental import pallas as pl
from jax.experimental.pallas import tpu as pltpu

MAP_UNITS = 2048
IN_CH = 256
BATCH = 8
DIST_THRESHOLD = 200.0
ROW_BLK = 256
N_BLOCKS = MAP_UNITS // ROW_BLK


def _fused_kernel(d_ref, x_ref, wc_ref, b_ref, y_ref,
                  edge_sc, x_sc, deg_sc, u_sc, z_sc):
    p = pl.program_id(0)
    j = pl.program_id(1)

    @pl.when(p == 0)
    def _phase0():
        d = d_ref[...]
        within = (d > 0.0) & (d < DIST_THRESHOLD)
        edge = jnp.where(within, d, 0.0)
        deg_sc[0, pl.ds(j * ROW_BLK, ROW_BLK)] = jnp.sum(edge, axis=1)
        edge_sc[pl.ds(j * ROW_BLK, ROW_BLK), :] = edge.astype(jnp.bfloat16)

        x = x_ref[...]
        x_sc[:, pl.ds(j * ROW_BLK, ROW_BLK), :] = x
        wc = wc_ref[...]
        zu = jax.lax.dot_general(
            x, wc, (((2,), (0,)), ((), ())),
            preferred_element_type=jnp.float32)
        u_sc[:, pl.ds(j * ROW_BLK, ROW_BLK)] = zu[:, :, 0]
        z_sc[:, pl.ds(j * ROW_BLK, ROW_BLK)] = zu[:, :, 1]

    @pl.when(p == 1)
    def _phase1():
        deg = deg_sc[...]
        dinv = jnp.where(deg > 0.0, jax.lax.rsqrt(deg), 0.0)
        w = (z_sc[...] * dinv).astype(jnp.bfloat16)
        edge = edge_sc[pl.ds(j * ROW_BLK, ROW_BLK), :]
        t = jax.lax.dot_general(
            w, edge, (((1,), (1,)), ((), ())),
            preferred_element_type=jnp.float32)
        deg_n = deg_sc[0, pl.ds(j * ROW_BLK, ROW_BLK)]
        dinv_n = jnp.where(deg_n > 0.0, jax.lax.rsqrt(deg_n), 0.0)
        u = u_sc[:, pl.ds(j * ROW_BLK, ROW_BLK)]
        out = u - dinv_n[None, :] * t + b_ref[0, 0]
        gcn = jax.nn.sigmoid(out)

        y_ref[:, :, 0:IN_CH] = x_sc[:, pl.ds(j * ROW_BLK, ROW_BLK), :]
        y_ref[:, :, IN_CH:IN_CH + 1] = gcn[:, :, None]


@jax.jit
def kernel(x, dist_mat, W, b):
    wc = jnp.concatenate([W[0], W[1]], axis=1)
    b2 = jnp.reshape(b, (1, 1)).astype(jnp.float32)

    y = pl.pallas_call(
        _fused_kernel,
        grid=(2, N_BLOCKS),
        in_specs=[
            pl.BlockSpec((ROW_BLK, MAP_UNITS), lambda p, j: (j * (1 - p), 0)),
            pl.BlockSpec((BATCH, ROW_BLK, IN_CH),
                         lambda p, j: (0, j * (1 - p), 0)),
            pl.BlockSpec((IN_CH, 2), lambda p, j: (0, 0)),
            pl.BlockSpec((1, 1), lambda p, j: (0, 0)),
        ],
        out_specs=pl.BlockSpec((BATCH, ROW_BLK, IN_CH + 1),
                               lambda p, j: (0, j * p, 0)),
        out_shape=jax.ShapeDtypeStruct(
            (BATCH, MAP_UNITS, IN_CH + 1), jnp.float32),
        scratch_shapes=[
            pltpu.VMEM((MAP_UNITS, MAP_UNITS), jnp.bfloat16),
            pltpu.VMEM((BATCH, MAP_UNITS, IN_CH), jnp.float32),
            pltpu.VMEM((1, MAP_UNITS), jnp.float32),
            pltpu.VMEM((BATCH, MAP_UNITS), jnp.float32),
            pltpu.VMEM((BATCH, MAP_UNITS), jnp.float32),
        ],
    )(dist_mat, x, wc, b2)

    return y


# speedup vs baseline: 2.0413x; 1.0286x over previous
import jax
import jax.numpy as jnp
from jax.experimental import pallas as pl
from jax.experimental.pallas import tpu as pltpu

MAP_UNITS = 2048
IN_CH = 256
BATCH = 8
DIST_THRESHOLD = 200.0
ROW_BLK = 256
N_BLOCKS = MAP_UNITS // ROW_BLK


def _fused_kernel(d_ref, x_ref, wc_ref, b_ref, y_ref,
                  edge_sc, x_sc, deg_sc, u_sc, z_sc):
    p = pl.program_id(0)
    j = pl.program_id(1)

    @pl.when(p == 0)
    def _phase0():
        d = d_ref[...]
        edge = jnp.where(d < DIST_THRESHOLD, d, 0.0)
        ones = jnp.ones((MAP_UNITS, 1), dtype=jnp.float32)
        deg_blk = jax.lax.dot_general(
            edge, ones, (((1,), (0,)), ((), ())),
            preferred_element_type=jnp.float32)
        deg_sc[0, pl.ds(j * ROW_BLK, ROW_BLK)] = deg_blk[:, 0]
        edge_sc[pl.ds(j * ROW_BLK, ROW_BLK), :] = edge.astype(jnp.bfloat16)

        x = x_ref[...]
        x_sc[:, pl.ds(j * ROW_BLK, ROW_BLK), :] = x
        wc = wc_ref[...]
        zu = jax.lax.dot_general(
            x, wc, (((2,), (0,)), ((), ())),
            preferred_element_type=jnp.float32)
        u_sc[:, pl.ds(j * ROW_BLK, ROW_BLK)] = zu[:, :, 0]
        z_sc[:, pl.ds(j * ROW_BLK, ROW_BLK)] = zu[:, :, 1]

    @pl.when(p == 1)
    def _phase1():
        deg = deg_sc[...]
        dinv = jnp.where(deg > 0.0, jax.lax.rsqrt(deg), 0.0)
        w = (z_sc[...] * dinv).astype(jnp.bfloat16)
        edge = edge_sc[pl.ds(j * ROW_BLK, ROW_BLK), :]
        t = jax.lax.dot_general(
            w, edge, (((1,), (1,)), ((), ())),
            preferred_element_type=jnp.float32)
        deg_n = deg_sc[0, pl.ds(j * ROW_BLK, ROW_BLK)]
        dinv_n = jnp.where(deg_n > 0.0, jax.lax.rsqrt(deg_n), 0.0)
        u = u_sc[:, pl.ds(j * ROW_BLK, ROW_BLK)]
        out = u - dinv_n[None, :] * t + b_ref[0, 0]
        gcn = jax.nn.sigmoid(out)

        y_ref[:, :, 0:IN_CH] = x_sc[:, pl.ds(j * ROW_BLK, ROW_BLK), :]
        y_ref[:, :, IN_CH:IN_CH + 1] = gcn[:, :, None]


@jax.jit
def kernel(x, dist_mat, W, b):
    wc = jnp.concatenate([W[0], W[1]], axis=1)
    b2 = jnp.reshape(b, (1, 1)).astype(jnp.float32)

    y = pl.pallas_call(
        _fused_kernel,
        grid=(2, N_BLOCKS),
        in_specs=[
            pl.BlockSpec((ROW_BLK, MAP_UNITS),
                         lambda p, j: (j * (1 - p) + (N_BLOCKS - 1) * p, 0)),
            pl.BlockSpec((BATCH, ROW_BLK, IN_CH),
                         lambda p, j: (0, j * (1 - p) + (N_BLOCKS - 1) * p, 0)),
            pl.BlockSpec((IN_CH, 2), lambda p, j: (0, 0)),
            pl.BlockSpec((1, 1), lambda p, j: (0, 0)),
        ],
        out_specs=pl.BlockSpec((BATCH, ROW_BLK, IN_CH + 1),
                               lambda p, j: (0, j * p, 0)),
        out_shape=jax.ShapeDtypeStruct(
            (BATCH, MAP_UNITS, IN_CH + 1), jnp.float32),
        scratch_shapes=[
            pltpu.VMEM((MAP_UNITS, MAP_UNITS), jnp.bfloat16),
            pltpu.VMEM((BATCH, MAP_UNITS, IN_CH), jnp.float32),
            pltpu.VMEM((1, MAP_UNITS), jnp.float32),
            pltpu.VMEM((BATCH, MAP_UNITS), jnp.float32),
            pltpu.VMEM((BATCH, MAP_UNITS), jnp.float32),
        ],
    )(dist_mat, x, wc, b2)

    return y
